# Initial kernel scaffold; baseline (speedup 1.0000x reference)
import jax
import jax.numpy as jnp
from jax.experimental import pallas as pl


def kernel(lf_list, lf_seg, batch_indices):
    raise NotImplementedError("write your pallas kernel here")


# speedup vs baseline: 2.8073x; 2.8073x over previous
import functools

import jax
import jax.numpy as jnp
from jax import lax
from jax.experimental import pallas as pl
from jax.experimental.pallas import tpu as pltpu
from jax.experimental.pallas import tpu_sc as plsc

_K = 4096
_D = 256
_B = 16384


@functools.cache
def _make_kernel(B, D):
    info = plsc.get_sparse_core_info()
    NC, NS = info.num_cores, info.num_subcores
    NW = NC * NS
    b_per_w = B // NW
    CH = 128
    n_ch = b_per_w // CH

    mesh = plsc.VectorSubcoreMesh(core_axis_name="c", subcore_axis_name="s")

    @functools.partial(
        pl.kernel,
        mesh=mesh,
        out_type=jax.ShapeDtypeStruct((B, D), jnp.float32),
        scratch_types=[
            pltpu.VMEM((n_ch, CH), jnp.int32),
            pltpu.VMEM((n_ch, CH), jnp.int32),
            pltpu.VMEM((2, CH, D), jnp.float32),
            pltpu.SemaphoreType.DMA,
            pltpu.SemaphoreType.DMA,
        ],
    )
    def k(lf_hbm, seg_hbm, bidx_hbm, out_hbm, bidx_v, segid_v, rows_v, isem, gsem):
        wid = lax.axis_index("s") * NC + lax.axis_index("c")
        base = wid * b_per_w

        pltpu.sync_copy(bidx_hbm.at[wid], bidx_v)

        cps = [
            pltpu.async_copy(seg_hbm.at[bidx_v.at[j]], segid_v.at[j], isem)
            for j in range(n_ch)
        ]
        for cp in cps:
            cp.wait()

        g = [None] * n_ch
        g[0] = pltpu.async_copy(lf_hbm.at[segid_v.at[0]], rows_v.at[0], gsem)
        for j in range(n_ch):
            if j + 1 < n_ch:
                g[j + 1] = pltpu.async_copy(
                    lf_hbm.at[segid_v.at[j + 1]], rows_v.at[(j + 1) % 2], gsem
                )
            g[j].wait()
            pltpu.sync_copy(
                rows_v.at[j % 2], out_hbm.at[pl.ds(base + j * CH, CH)]
            )

    return k


def kernel(lf_list, lf_seg, batch_indices):
    info = plsc.get_sparse_core_info()
    NW = info.num_cores * info.num_subcores
    B, D = batch_indices.shape[0], lf_list.shape[1]
    seg_flat = lf_seg.reshape(-1)
    b_per_w = B // NW
    bidx3 = batch_indices.reshape(NW, b_per_w // 128, 128)
    return _make_kernel(B, D)(lf_list, seg_flat, bidx3)
